# Initial kernel scaffold; baseline (speedup 1.0000x reference)
import jax
import jax.numpy as jnp
from jax.experimental import pallas as pl


def kernel(id_tensor, W):
    raise NotImplementedError("write your pallas kernel here")


# speedup vs baseline: 1.6860x; 1.6860x over previous
import functools

import jax
import jax.numpy as jnp
from jax import lax
from jax.experimental import pallas as pl
from jax.experimental.pallas import tpu as pltpu
from jax.experimental.pallas import tpu_sc as plsc

_NUM_CORES = 2
_NUM_SUBCORES = 16
_NW = _NUM_CORES * _NUM_SUBCORES
_C = 128


@functools.lru_cache(maxsize=None)
def _gather_kernel(B, D):
    n_per_w = B // _NW
    n_chunks = n_per_w // _C
    mesh = plsc.VectorSubcoreMesh(core_axis_name="c", subcore_axis_name="s")

    @functools.partial(
        pl.kernel,
        out_type=jax.ShapeDtypeStruct((_NW, n_chunks, _C, D), jnp.float32),
        mesh=mesh,
        scratch_types=[
            pltpu.VMEM((n_chunks, _C), jnp.int32),
            pltpu.VMEM((_C, D), jnp.float32),
            pltpu.SemaphoreType.DMA,
        ],
        compiler_params=pltpu.CompilerParams(use_tc_tiling_on_sc=False),
    )
    def k(idx_hbm, table_hbm, out_hbm, idx_v, rows_v, sem):
        wid = lax.axis_index("s") * _NUM_CORES + lax.axis_index("c")
        pltpu.sync_copy(idx_hbm.at[wid], idx_v)

        def body(j, carry):
            pltpu.async_copy(table_hbm.at[idx_v.at[j]], rows_v, sem).wait()
            pltpu.sync_copy(rows_v, out_hbm.at[wid, j])
            return carry

        lax.fori_loop(0, n_chunks, body, 0)

    return k


def kernel(id_tensor, W):
    S0, S1 = id_tensor.shape
    B = S0 * S1
    D = W.shape[1]
    idx = id_tensor.reshape(_NW, (B // _NW) // _C, _C)
    out = _gather_kernel(B, D)(idx, W)
    return out.reshape(S0, S1, D)


# speedup vs baseline: 1.8760x; 1.1127x over previous
import functools

import jax
import jax.numpy as jnp
from jax import lax
from jax.experimental import pallas as pl
from jax.experimental.pallas import tpu as pltpu
from jax.experimental.pallas import tpu_sc as plsc

_NUM_CORES = 2
_NUM_SUBCORES = 16
_NW = _NUM_CORES * _NUM_SUBCORES
_C = 128
_K = 4


@functools.lru_cache(maxsize=None)
def _gather_kernel(B, D):
    n_per_w = B // _NW
    n_chunks = n_per_w // _C
    n_groups = n_chunks // _K
    mesh = plsc.VectorSubcoreMesh(core_axis_name="c", subcore_axis_name="s")

    @functools.partial(
        pl.kernel,
        out_type=jax.ShapeDtypeStruct((_NW, n_groups, _K, _C, D), jnp.float32),
        mesh=mesh,
        scratch_types=[
            pltpu.VMEM((n_chunks, _C), jnp.int32),
            pltpu.VMEM((2, _K, _C, D), jnp.float32),
            pltpu.SemaphoreType.DMA((2,)),
            pltpu.SemaphoreType.DMA((2,)),
        ],
        compiler_params=pltpu.CompilerParams(use_tc_tiling_on_sc=False),
    )
    def k(idx_hbm, table_hbm, out_hbm, idx_v, rows_v, gsem, osem):
        wid = lax.axis_index("s") * _NUM_CORES + lax.axis_index("c")
        pltpu.sync_copy(idx_hbm.at[wid], idx_v)

        def fire_gathers(g, p):
            for kk in range(_K):
                pltpu.async_copy(
                    table_hbm.at[idx_v.at[g * _K + kk]],
                    rows_v.at[p, kk],
                    gsem.at[p],
                )

        def drain_gathers(g, p):
            for kk in range(_K):
                pltpu.make_async_copy(
                    table_hbm.at[idx_v.at[g * _K + kk]],
                    rows_v.at[p, kk],
                    gsem.at[p],
                ).wait()

        def fire_out(g, p):
            pltpu.async_copy(rows_v.at[p], out_hbm.at[wid, g], osem.at[p])

        def wait_out(g, p):
            pltpu.make_async_copy(rows_v.at[p], out_hbm.at[wid, g], osem.at[p]).wait()

        fire_gathers(0, 0)
        fire_gathers(1, 1)
        drain_gathers(0, 0)
        fire_out(0, 0)

        def body(i, carry):
            for p2 in range(2):
                g = 1 + i * 2 + p2
                p = (1 + p2) % 2
                nxt = 1 - p
                wait_out(g - 1, nxt)
                fire_gathers(g + 1, nxt)
                drain_gathers(g, p)
                fire_out(g, p)
            return carry

        lax.fori_loop(0, (n_groups - 2) // 2, body, 0)

        g_last = n_groups - 1
        drain_gathers(g_last, 1)
        fire_out(g_last, 1)
        wait_out(g_last - 1, 0)
        wait_out(g_last, 1)

    return k


def kernel(id_tensor, W):
    S0, S1 = id_tensor.shape
    B = S0 * S1
    D = W.shape[1]
    idx = id_tensor.reshape(_NW, (B // _NW) // _C, _C)
    out = _gather_kernel(B, D)(idx, W)
    return out.reshape(S0, S1, D)
